# Initial kernel scaffold; baseline (speedup 1.0000x reference)
import jax
import jax.numpy as jnp
from jax.experimental import pallas as pl


def kernel(x, edge_index_follows, edge_weight_follows, edge_index_likes, edge_weight_likes, W0_follows, b0_follows, W0_likes, b0_likes, W1_follows, b1_follows, W1_likes, b1_likes):
    raise NotImplementedError("write your pallas kernel here")


# speedup vs baseline: 4.8969x; 4.8969x over previous
import functools
import jax
import jax.numpy as jnp
from jax import lax
from jax.experimental import pallas as pl
from jax.experimental.pallas import tpu as pltpu
from jax.experimental.pallas import tpu_sc as plsc

N = 10000
D = 128
E = 160000
NC = 2
NS = 16
NP = 10240
NPT = NP // NS
EPT = E // NS
CK = 80
NCH = EPT // CK
RB = NPT // CK
def _rsqrt16(v):
    s = (v + 1.0) * 0.5
    for _ in range(16):
        s = 0.5 * (s + v / s)
    return 1.0 / s


def _sc_body(x_hbm, srcs_hbm, dsts_hbm, ws_hbm, ys_hbm, g_hbm,
             src_ck, dst_ck, w_ck, rows_vm, m_vm, q_vm,
             ga_vm, gb_vm, ones_vm, tmp_vm, a_sl, b_sl,
             y_sh, dego_sh, degi_sh, p_sh, a_sh, b_sh):
    cid = lax.axis_index("c")
    sid = lax.axis_index("s")
    r0 = sid * NPT

    z16 = jnp.zeros((16,), jnp.float32)

    def _zrow(j, _):
        for c in range(D // 16):
            rows_vm[j, pl.ds(c * 16, 16)] = z16
        return 0
    lax.fori_loop(0, CK, _zrow, 0)

    def _ztmp(i, _):
        tmp_vm[pl.ds(i * 16, 16)] = z16
        return 0
    lax.fori_loop(0, NPT // 16, _ztmp, 0)
    for i in range(CK // 16):
        ones_vm[pl.ds(i * 16, 16)] = jnp.full((16,), 1.0, jnp.float32)

    for rb in range(RB):
        pltpu.sync_copy(rows_vm, y_sh.at[pl.ds(r0 + rb * CK, CK), :])
    pltpu.sync_copy(tmp_vm, dego_sh.at[pl.ds(r0, NPT)])
    pltpu.sync_copy(tmp_vm, degi_sh.at[pl.ds(r0, NPT)])
    pltpu.sync_copy(tmp_vm, p_sh.at[pl.ds(r0, NPT)])

    plsc.subcore_barrier()

    def _deg(j, _):
        pltpu.sync_copy(srcs_hbm.at[cid, sid, j], src_ck)
        pltpu.sync_copy(dsts_hbm.at[cid, sid, j], dst_ck)
        pltpu.sync_copy(ones_vm, dego_sh.at[src_ck], add=True)
        pltpu.sync_copy(ones_vm, degi_sh.at[dst_ck], add=True)
        return 0
    lax.fori_loop(0, NCH, _deg, 0)

    plsc.subcore_barrier()

    pltpu.sync_copy(dego_sh.at[pl.ds(r0, NPT)], tmp_vm)

    def _ra(i, _):
        v = jnp.maximum(tmp_vm[pl.ds(i * 16, 16)], 1.0)
        tmp_vm[pl.ds(i * 16, 16)] = _rsqrt16(v)
        return 0
    lax.fori_loop(0, NPT // 16, _ra, 0)
    pltpu.sync_copy(tmp_vm, a_sh.at[pl.ds(r0, NPT)])

    pltpu.sync_copy(degi_sh.at[pl.ds(r0, NPT)], tmp_vm)
    lax.fori_loop(0, NPT // 16, _ra, 0)
    pltpu.sync_copy(tmp_vm, b_sh.at[pl.ds(r0, NPT)])

    plsc.subcore_barrier()

    def _edge(j, _):
        pltpu.sync_copy(srcs_hbm.at[cid, sid, j], src_ck)
        pltpu.sync_copy(dsts_hbm.at[cid, sid, j], dst_ck)
        pltpu.sync_copy(ws_hbm.at[cid, sid, j], w_ck)

        pltpu.sync_copy(a_sh.at[src_ck], ga_vm)
        pltpu.sync_copy(b_sh.at[dst_ck], gb_vm)
        for k in range(CK // 16):
            sl = pl.ds(k * 16, 16)
            wv = w_ck[sl]
            m_vm[sl] = wv * ga_vm[sl]
            q_vm[sl] = wv * gb_vm[sl]
        pltpu.sync_copy(q_vm, p_sh.at[src_ck], add=True)

        pltpu.sync_copy(x_hbm.at[src_ck], rows_vm)

        def _scale(g, _):
            mv16 = m_vm[pl.ds(g * 16, 16)]
            for rr in range(16):
                r = g * 16 + rr
                mv = mv16[rr]
                for c in range(D // 16):
                    sl = pl.ds(c * 16, 16)
                    rows_vm[r, sl] = rows_vm[r, sl] * mv
            return 0
        lax.fori_loop(0, CK // 16, _scale, 0)

        pltpu.sync_copy(rows_vm, y_sh.at[dst_ck], add=True)
        return 0
    lax.fori_loop(0, NCH, _edge, 0)

    plsc.subcore_barrier()

    pltpu.sync_copy(a_sh.at[pl.ds(r0, NPT)], a_sl)
    pltpu.sync_copy(b_sh.at[pl.ds(r0, NPT)], b_sl)

    def _out_block(rb, _):
        base = rb * CK
        pltpu.sync_copy(y_sh.at[pl.ds(r0 + base, CK), :], rows_vm)

        def _brow(g, _):
            bv16 = b_sl[pl.ds(base + g * 16, 16)]
            for rr in range(16):
                r = g * 16 + rr
                bv = bv16[rr]
                for c in range(D // 16):
                    sl = pl.ds(c * 16, 16)
                    rows_vm[r, sl] = rows_vm[r, sl] * bv
            return 0
        lax.fori_loop(0, CK // 16, _brow, 0)
        pltpu.sync_copy(rows_vm, ys_hbm.at[cid, pl.ds(r0 + base, CK), :])
        return 0
    lax.fori_loop(0, RB, _out_block, 0)

    pltpu.sync_copy(p_sh.at[pl.ds(r0, NPT)], tmp_vm)

    def _g(i, _):
        sl = pl.ds(i * 16, 16)
        tmp_vm[sl] = tmp_vm[sl] * a_sl[sl]
        return 0
    lax.fori_loop(0, NPT // 16, _g, 0)
    pltpu.sync_copy(tmp_vm, g_hbm.at[cid, pl.ds(r0, NPT)])


_sc_call = functools.partial(
    pl.kernel,
    out_type=(
        jax.ShapeDtypeStruct((NC, NP, D), jnp.float32),
        jax.ShapeDtypeStruct((NC, NP), jnp.float32),
    ),
    mesh=plsc.VectorSubcoreMesh(core_axis_name="c", subcore_axis_name="s"),
    scratch_types=[
        pltpu.VMEM((CK,), jnp.int32),
        pltpu.VMEM((CK,), jnp.int32),
        pltpu.VMEM((CK,), jnp.float32),
        pltpu.VMEM((CK, D), jnp.float32),
        pltpu.VMEM((CK,), jnp.float32),
        pltpu.VMEM((CK,), jnp.float32),
        pltpu.VMEM((CK,), jnp.float32),
        pltpu.VMEM((CK,), jnp.float32),
        pltpu.VMEM((CK,), jnp.float32),
        pltpu.VMEM((NPT,), jnp.float32),
        pltpu.VMEM((NPT,), jnp.float32),
        pltpu.VMEM((NPT,), jnp.float32),
        pltpu.VMEM_SHARED((NP, D), jnp.float32),
        pltpu.VMEM_SHARED((NP,), jnp.float32),
        pltpu.VMEM_SHARED((NP,), jnp.float32),
        pltpu.VMEM_SHARED((NP,), jnp.float32),
        pltpu.VMEM_SHARED((NP,), jnp.float32),
        pltpu.VMEM_SHARED((NP,), jnp.float32),
    ],
)(_sc_body)


BLK = 256
NBLK = NP // BLK


def _tc_body(ysf, ysl, gf, gl, w0f, w0l, b0f, b0l, w1f, w1l, b1f, b1l,
             out, acc):
    i = pl.program_id(0)

    @pl.when(i == 0)
    def _():
        acc[...] = jnp.zeros((2, D), jnp.float32)

    h = jnp.maximum(
        jnp.dot(ysf[...], w0f[...], preferred_element_type=jnp.float32)
        + jnp.dot(ysl[...], w0l[...], preferred_element_type=jnp.float32)
        + b0f[...] + b0l[...],
        0.0,
    )
    acc[0:1, :] += jnp.dot(gf[0], h, preferred_element_type=jnp.float32)
    acc[1:2, :] += jnp.dot(gl[0], h, preferred_element_type=jnp.float32)

    @pl.when(i == NBLK - 1)
    def _():
        out[...] = (
            jnp.dot(acc[0:1, :] * (1.0 / N), w1f[...],
                    preferred_element_type=jnp.float32)
            + jnp.dot(acc[1:2, :] * (1.0 / N), w1l[...],
                      preferred_element_type=jnp.float32)
            + b1f[...] + b1l[...]
        )


_tc_call = pl.pallas_call(
    _tc_body,
    grid=(NBLK,),
    in_specs=[
        pl.BlockSpec((BLK, D), lambda i: (i, 0)),
        pl.BlockSpec((BLK, D), lambda i: (i, 0)),
        pl.BlockSpec((1, 1, BLK), lambda i: (i, 0, 0)),
        pl.BlockSpec((1, 1, BLK), lambda i: (i, 0, 0)),
        pl.BlockSpec((D, D), lambda i: (0, 0)),
        pl.BlockSpec((D, D), lambda i: (0, 0)),
        pl.BlockSpec((1, D), lambda i: (0, 0)),
        pl.BlockSpec((1, D), lambda i: (0, 0)),
        pl.BlockSpec((D, D), lambda i: (0, 0)),
        pl.BlockSpec((D, D), lambda i: (0, 0)),
        pl.BlockSpec((1, D), lambda i: (0, 0)),
        pl.BlockSpec((1, D), lambda i: (0, 0)),
    ],
    out_specs=pl.BlockSpec((1, D), lambda i: (0, 0)),
    out_shape=jax.ShapeDtypeStruct((1, D), jnp.float32),
    scratch_shapes=[pltpu.VMEM((2, D), jnp.float32)],
)


def kernel(x, edge_index_follows, edge_weight_follows, edge_index_likes,
           edge_weight_likes, W0_follows, b0_follows, W0_likes, b0_likes,
           W1_follows, b1_follows, W1_likes, b1_likes):
    srcs = jnp.stack([edge_index_follows[0], edge_index_likes[0]]
                     ).reshape(NC, NS, NCH, CK)
    dsts = jnp.stack([edge_index_follows[1], edge_index_likes[1]]
                     ).reshape(NC, NS, NCH, CK)
    ws = jnp.stack([edge_weight_follows, edge_weight_likes]
                   ).reshape(NC, NS, NCH, CK)

    ys, g = _sc_call(x, srcs, dsts, ws)

    out = _tc_call(
        ys[0], ys[1],
        g[0].reshape(NBLK, 1, BLK), g[1].reshape(NBLK, 1, BLK),
        W0_follows, W0_likes,
        b0_follows.reshape(1, D), b0_likes.reshape(1, D),
        W1_follows, W1_likes,
        b1_follows.reshape(1, D), b1_likes.reshape(1, D),
    )
    return out


# speedup vs baseline: 11.3135x; 2.3104x over previous
import functools
import jax
import jax.numpy as jnp
from jax import lax
from jax.experimental import pallas as pl
from jax.experimental.pallas import tpu as pltpu
from jax.experimental.pallas import tpu_sc as plsc

N = 10000
D = 128
E = 160000
NC = 2
NS = 16
NP = 10240
NPT = NP // NS
EPT = E // NS
CK = 80
NCH = EPT // CK
CPS = 25
NSLAB = NCH // CPS
RB = NPT // CK
def _rsqrt16(v):
    s = (v + 1.0) * 0.5
    for _ in range(16):
        s = 0.5 * (s + v / s)
    return 1.0 / s


def _sc_body(x_hbm, srcs_hbm, dsts_hbm, ws_hbm, ys_hbm, g_hbm,
             src_g, dst_g, w_g, ga_g, gb_g, rows_a, rows_b,
             ones_vm, tmp_vm, a_sl, b_sl,
             y_sh, dego_sh, degi_sh, p_sh, a_sh, b_sh,
             sem_g, sem_s, sem_q):
    cid = lax.axis_index("c")
    sid = lax.axis_index("s")
    r0 = sid * NPT

    z16 = jnp.zeros((16,), jnp.float32)

    def _zrow(j, _):
        for c in range(D // 16):
            rows_a[j, pl.ds(c * 16, 16)] = z16
        return 0
    lax.fori_loop(0, CK, _zrow, 0)

    def _ztmp(i, _):
        tmp_vm[pl.ds(i * 16, 16)] = z16
        return 0
    lax.fori_loop(0, NPT // 16, _ztmp, 0)
    for i in range(CK // 16):
        ones_vm[pl.ds(i * 16, 16)] = jnp.full((16,), 1.0, jnp.float32)

    for rb in range(RB):
        pltpu.sync_copy(rows_a, y_sh.at[pl.ds(r0 + rb * CK, CK), :])
    pltpu.sync_copy(tmp_vm, dego_sh.at[pl.ds(r0, NPT)])
    pltpu.sync_copy(tmp_vm, degi_sh.at[pl.ds(r0, NPT)])
    pltpu.sync_copy(tmp_vm, p_sh.at[pl.ds(r0, NPT)])

    plsc.subcore_barrier()

    def _deg(s_, _):
        pltpu.sync_copy(srcs_hbm.at[cid, sid, s_], src_g)
        pltpu.sync_copy(dsts_hbm.at[cid, sid, s_], dst_g)
        descs = []
        for i in range(CPS):
            descs.append(pltpu.async_copy(
                ones_vm, dego_sh.at[src_g.at[i]], sem_q, add=True))
            descs.append(pltpu.async_copy(
                ones_vm, degi_sh.at[dst_g.at[i]], sem_q, add=True))
        for d in descs:
            d.wait()
        return 0
    lax.fori_loop(0, NSLAB, _deg, 0)

    plsc.subcore_barrier()

    pltpu.sync_copy(dego_sh.at[pl.ds(r0, NPT)], tmp_vm)

    def _ra(i, _):
        v = jnp.maximum(tmp_vm[pl.ds(i * 16, 16)], 1.0)
        tmp_vm[pl.ds(i * 16, 16)] = _rsqrt16(v)
        return 0
    lax.fori_loop(0, NPT // 16, _ra, 0)
    pltpu.sync_copy(tmp_vm, a_sh.at[pl.ds(r0, NPT)])

    pltpu.sync_copy(degi_sh.at[pl.ds(r0, NPT)], tmp_vm)
    lax.fori_loop(0, NPT // 16, _ra, 0)
    pltpu.sync_copy(tmp_vm, b_sh.at[pl.ds(r0, NPT)])

    plsc.subcore_barrier()

    def _scale_rows(rows_ref, ci):
        def _sg(g, _):
            mv16 = ga_g[ci, pl.ds(g * 16, 16)]
            for rr in range(16):
                r = g * 16 + rr
                mv = mv16[rr]
                for c in range(D // 16):
                    sl = pl.ds(c * 16, 16)
                    rows_ref[r, sl] = rows_ref[r, sl] * mv
            return 0
        lax.fori_loop(0, CK // 16, _sg, 0)

    def _slab(s_, _):
        pltpu.sync_copy(srcs_hbm.at[cid, sid, s_], src_g)
        pltpu.sync_copy(dsts_hbm.at[cid, sid, s_], dst_g)
        pltpu.sync_copy(ws_hbm.at[cid, sid, s_], w_g)

        descs = []
        for i in range(CPS):
            descs.append(pltpu.async_copy(
                a_sh.at[src_g.at[i]], ga_g.at[i], sem_g))
            descs.append(pltpu.async_copy(
                b_sh.at[dst_g.at[i]], gb_g.at[i], sem_g))
        for d in descs:
            d.wait()

        def _mq(i, _):
            for k in range(CK // 16):
                sl = pl.ds(k * 16, 16)
                wv = w_g[i, sl]
                ga_g[i, sl] = wv * ga_g[i, sl]
                gb_g[i, sl] = wv * gb_g[i, sl]
            return 0
        lax.fori_loop(0, CPS, _mq, 0)

        descs = []
        for i in range(CPS):
            descs.append(pltpu.async_copy(
                gb_g.at[i], p_sh.at[src_g.at[i]], sem_q, add=True))
        for d in descs:
            d.wait()

        def _wave(t, _):
            ca = 2 * t
            cb = 2 * t + 1
            dga = pltpu.async_copy(x_hbm.at[src_g.at[ca]], rows_a, sem_g)
            dgb = pltpu.async_copy(x_hbm.at[src_g.at[cb]], rows_b, sem_g)
            dga.wait()
            _scale_rows(rows_a, ca)
            dsa = pltpu.async_copy(
                rows_a, y_sh.at[dst_g.at[ca]], sem_s, add=True)
            dgb.wait()
            _scale_rows(rows_b, cb)
            dsb = pltpu.async_copy(
                rows_b, y_sh.at[dst_g.at[cb]], sem_s, add=True)
            dsa.wait()
            dsb.wait()
            return 0
        lax.fori_loop(0, CPS // 2, _wave, 0)

        ct = CPS - 1
        pltpu.async_copy(x_hbm.at[src_g.at[ct]], rows_a, sem_g).wait()
        _scale_rows(rows_a, ct)
        pltpu.async_copy(
            rows_a, y_sh.at[dst_g.at[ct]], sem_s, add=True).wait()
        return 0
    lax.fori_loop(0, NSLAB, _slab, 0)

    plsc.subcore_barrier()

    pltpu.sync_copy(a_sh.at[pl.ds(r0, NPT)], a_sl)
    pltpu.sync_copy(b_sh.at[pl.ds(r0, NPT)], b_sl)

    def _out_block(rb, _):
        base = rb * CK
        pltpu.sync_copy(y_sh.at[pl.ds(r0 + base, CK), :], rows_a)

        def _brow(g, _):
            bv16 = b_sl[pl.ds(base + g * 16, 16)]
            for rr in range(16):
                r = g * 16 + rr
                bv = bv16[rr]
                for c in range(D // 16):
                    sl = pl.ds(c * 16, 16)
                    rows_a[r, sl] = rows_a[r, sl] * bv
            return 0
        lax.fori_loop(0, CK // 16, _brow, 0)
        pltpu.sync_copy(rows_a, ys_hbm.at[cid, pl.ds(r0 + base, CK), :])
        return 0
    lax.fori_loop(0, RB, _out_block, 0)

    pltpu.sync_copy(p_sh.at[pl.ds(r0, NPT)], tmp_vm)

    def _g(i, _):
        sl = pl.ds(i * 16, 16)
        tmp_vm[sl] = tmp_vm[sl] * a_sl[sl]
        return 0
    lax.fori_loop(0, NPT // 16, _g, 0)
    pltpu.sync_copy(tmp_vm, g_hbm.at[cid, pl.ds(r0, NPT)])


_sc_call = functools.partial(
    pl.kernel,
    out_type=(
        jax.ShapeDtypeStruct((NC, NP, D), jnp.float32),
        jax.ShapeDtypeStruct((NC, NP), jnp.float32),
    ),
    mesh=plsc.VectorSubcoreMesh(core_axis_name="c", subcore_axis_name="s"),
    scratch_types=[
        pltpu.VMEM((CPS, CK), jnp.int32),
        pltpu.VMEM((CPS, CK), jnp.int32),
        pltpu.VMEM((CPS, CK), jnp.float32),
        pltpu.VMEM((CPS, CK), jnp.float32),
        pltpu.VMEM((CPS, CK), jnp.float32),
        pltpu.VMEM((CK, D), jnp.float32),
        pltpu.VMEM((CK, D), jnp.float32),
        pltpu.VMEM((CK,), jnp.float32),
        pltpu.VMEM((NPT,), jnp.float32),
        pltpu.VMEM((NPT,), jnp.float32),
        pltpu.VMEM((NPT,), jnp.float32),
        pltpu.VMEM_SHARED((NP, D), jnp.float32),
        pltpu.VMEM_SHARED((NP,), jnp.float32),
        pltpu.VMEM_SHARED((NP,), jnp.float32),
        pltpu.VMEM_SHARED((NP,), jnp.float32),
        pltpu.VMEM_SHARED((NP,), jnp.float32),
        pltpu.VMEM_SHARED((NP,), jnp.float32),
        pltpu.SemaphoreType.DMA,
        pltpu.SemaphoreType.DMA,
        pltpu.SemaphoreType.DMA,
    ],
)(_sc_body)


BLK = 256
NBLK = NP // BLK


def _tc_body(ysf, ysl, gf, gl, w0f, w0l, b0f, b0l, w1f, w1l, b1f, b1l,
             out, acc):
    i = pl.program_id(0)

    @pl.when(i == 0)
    def _():
        acc[...] = jnp.zeros((2, D), jnp.float32)

    h = jnp.maximum(
        jnp.dot(ysf[...], w0f[...], preferred_element_type=jnp.float32)
        + jnp.dot(ysl[...], w0l[...], preferred_element_type=jnp.float32)
        + b0f[...] + b0l[...],
        0.0,
    )
    acc[0:1, :] += jnp.dot(gf[0], h, preferred_element_type=jnp.float32)
    acc[1:2, :] += jnp.dot(gl[0], h, preferred_element_type=jnp.float32)

    @pl.when(i == NBLK - 1)
    def _():
        out[...] = (
            jnp.dot(acc[0:1, :] * (1.0 / N), w1f[...],
                    preferred_element_type=jnp.float32)
            + jnp.dot(acc[1:2, :] * (1.0 / N), w1l[...],
                      preferred_element_type=jnp.float32)
            + b1f[...] + b1l[...]
        )


_tc_call = pl.pallas_call(
    _tc_body,
    grid=(NBLK,),
    in_specs=[
        pl.BlockSpec((BLK, D), lambda i: (i, 0)),
        pl.BlockSpec((BLK, D), lambda i: (i, 0)),
        pl.BlockSpec((1, 1, BLK), lambda i: (i, 0, 0)),
        pl.BlockSpec((1, 1, BLK), lambda i: (i, 0, 0)),
        pl.BlockSpec((D, D), lambda i: (0, 0)),
        pl.BlockSpec((D, D), lambda i: (0, 0)),
        pl.BlockSpec((1, D), lambda i: (0, 0)),
        pl.BlockSpec((1, D), lambda i: (0, 0)),
        pl.BlockSpec((D, D), lambda i: (0, 0)),
        pl.BlockSpec((D, D), lambda i: (0, 0)),
        pl.BlockSpec((1, D), lambda i: (0, 0)),
        pl.BlockSpec((1, D), lambda i: (0, 0)),
    ],
    out_specs=pl.BlockSpec((1, D), lambda i: (0, 0)),
    out_shape=jax.ShapeDtypeStruct((1, D), jnp.float32),
    scratch_shapes=[pltpu.VMEM((2, D), jnp.float32)],
)


def kernel(x, edge_index_follows, edge_weight_follows, edge_index_likes,
           edge_weight_likes, W0_follows, b0_follows, W0_likes, b0_likes,
           W1_follows, b1_follows, W1_likes, b1_likes):
    srcs = jnp.stack([edge_index_follows[0], edge_index_likes[0]]
                     ).reshape(NC, NS, NSLAB, CPS, CK)
    dsts = jnp.stack([edge_index_follows[1], edge_index_likes[1]]
                     ).reshape(NC, NS, NSLAB, CPS, CK)
    ws = jnp.stack([edge_weight_follows, edge_weight_likes]
                   ).reshape(NC, NS, NSLAB, CPS, CK)

    ys, g = _sc_call(x, srcs, dsts, ws)

    out = _tc_call(
        ys[0], ys[1],
        g[0].reshape(NBLK, 1, BLK), g[1].reshape(NBLK, 1, BLK),
        W0_follows, W0_likes,
        b0_follows.reshape(1, D), b0_likes.reshape(1, D),
        W1_follows, W1_likes,
        b1_follows.reshape(1, D), b1_likes.reshape(1, D),
    )
    return out


# speedup vs baseline: 12.8199x; 1.1332x over previous
import functools
import jax
import jax.numpy as jnp
from jax import lax
from jax.experimental import pallas as pl
from jax.experimental.pallas import tpu as pltpu
from jax.experimental.pallas import tpu_sc as plsc

N = 10000
D = 128
E = 160000
NC = 2
NS = 16
NP = 10240
NPT = NP // NS
EPT = E // NS
CK = 80
NCH = EPT // CK
CPS = 25
NSLAB = NCH // CPS
RB = NPT // CK
def _rsqrt16(v):
    s = (v + 1.0) * 0.5
    for _ in range(16):
        s = 0.5 * (s + v / s)
    return 1.0 / s


def _sc_body(x_hbm, srcf_hbm, dstf_hbm, wf_hbm, srcl_hbm, dstl_hbm,
             wl_hbm, ys_hbm, g_hbm,
             src_g, dst_g, w_g, ga_g, rows_a, rows_b, rows_c,
             ones_vm, tmp_vm, b_sl,
             y_sh, p_sh, a_sh, b_sh,
             sem_ga, sem_gb, sem_gc, sem_sa, sem_sb, sem_sc, sem_q):

    def _load_slab(s_, want_w):
        cid_ = lax.axis_index("c")
        sid_ = lax.axis_index("s")

        @pl.when(cid_ == 0)
        def _():
            pltpu.sync_copy(srcf_hbm.at[sid_, s_], src_g)
            pltpu.sync_copy(dstf_hbm.at[sid_, s_], dst_g)
            if want_w:
                pltpu.sync_copy(wf_hbm.at[sid_, s_], w_g)

        @pl.when(cid_ == 1)
        def _():
            pltpu.sync_copy(srcl_hbm.at[sid_, s_], src_g)
            pltpu.sync_copy(dstl_hbm.at[sid_, s_], dst_g)
            if want_w:
                pltpu.sync_copy(wl_hbm.at[sid_, s_], w_g)
    cid = lax.axis_index("c")
    sid = lax.axis_index("s")
    r0 = sid * NPT

    z16 = jnp.zeros((16,), jnp.float32)

    def _zrow(j, _):
        for c in range(D // 16):
            rows_a[j, pl.ds(c * 16, 16)] = z16
        return 0
    lax.fori_loop(0, CK, _zrow, 0)

    def _ztmp(i, _):
        tmp_vm[pl.ds(i * 16, 16)] = z16
        return 0
    lax.fori_loop(0, NPT // 16, _ztmp, 0)
    for i in range(CK // 16):
        ones_vm[pl.ds(i * 16, 16)] = jnp.full((16,), 1.0, jnp.float32)

    for rb in range(RB):
        @pl.when(r0 + rb * CK < N)
        def _():
            pltpu.sync_copy(rows_a, y_sh.at[pl.ds(r0 + rb * CK, CK), :])
    pltpu.sync_copy(tmp_vm, a_sh.at[pl.ds(r0, NPT)])
    pltpu.sync_copy(tmp_vm, b_sh.at[pl.ds(r0, NPT)])
    pltpu.sync_copy(tmp_vm, p_sh.at[pl.ds(r0, NPT)])

    plsc.subcore_barrier()

    def _deg(s_, _):
        _load_slab(s_, False)
        descs = []
        for i in range(CPS):
            descs.append(pltpu.async_copy(
                ones_vm, a_sh.at[src_g.at[i]], sem_q, add=True))
            descs.append(pltpu.async_copy(
                ones_vm, b_sh.at[dst_g.at[i]], sem_q, add=True))
        for d in descs:
            d.wait()
        return 0
    lax.fori_loop(0, NSLAB, _deg, 0)

    plsc.subcore_barrier()

    pltpu.sync_copy(a_sh.at[pl.ds(r0, NPT)], tmp_vm)

    def _ra(i, _):
        v = jnp.maximum(tmp_vm[pl.ds(i * 16, 16)], 1.0)
        tmp_vm[pl.ds(i * 16, 16)] = _rsqrt16(v)
        return 0
    lax.fori_loop(0, NPT // 16, _ra, 0)
    pltpu.sync_copy(tmp_vm, a_sh.at[pl.ds(r0, NPT)])

    pltpu.sync_copy(b_sh.at[pl.ds(r0, NPT)], tmp_vm)
    lax.fori_loop(0, NPT // 16, _ra, 0)
    pltpu.sync_copy(tmp_vm, b_sh.at[pl.ds(r0, NPT)])

    plsc.subcore_barrier()

    def _scale_rows(rows_ref, ci):
        def _sg(g, _):
            mv16 = ga_g[ci, pl.ds(g * 16, 16)]
            for rr in range(16):
                r = g * 16 + rr
                mv = mv16[rr]
                for c in range(D // 16):
                    sl = pl.ds(c * 16, 16)
                    rows_ref[r, sl] = rows_ref[r, sl] * mv
            return 0
        lax.fori_loop(0, CK // 16, _sg, 0)

    def _fire_g(slot_ref, ci, sem):
        return pltpu.async_copy(x_hbm.at[src_g.at[ci]], slot_ref, sem)

    def _wait_g(slot_ref, sem):
        pltpu.make_async_copy(x_hbm.at[src_g.at[0]], slot_ref, sem).wait()

    def _fire_s(slot_ref, ci, sem):
        return pltpu.async_copy(
            slot_ref, y_sh.at[dst_g.at[ci]], sem, add=True)

    def _wait_s(slot_ref, sem):
        pltpu.make_async_copy(
            slot_ref, y_sh.at[dst_g.at[0]], sem).wait()

    def _slab(s_, _):
        _load_slab(s_, True)

        descs = []
        for i in range(CPS):
            descs.append(pltpu.async_copy(
                a_sh.at[src_g.at[i]], ga_g.at[i], sem_q))
            descs.append(pltpu.async_copy(
                b_sh.at[dst_g.at[i]], rows_a.at[i, pl.ds(0, CK)], sem_q))
        for d in descs:
            d.wait()

        def _mq(i, _):
            for k in range(CK // 16):
                sl = pl.ds(k * 16, 16)
                wv = w_g[i, sl]
                ga_g[i, sl] = wv * ga_g[i, sl]
                w_g[i, sl] = wv * rows_a[i, sl]
            return 0
        lax.fori_loop(0, CPS, _mq, 0)

        for i in range(CPS):
            pltpu.async_copy(w_g.at[i], p_sh.at[src_g.at[i]], sem_q,
                             add=True)

        _fire_g(rows_a, 0, sem_ga)
        _fire_g(rows_b, 1, sem_gb)
        _fire_g(rows_c, 2, sem_gc)

        def _body(t, _):
            ca = 3 * t
            _wait_g(rows_a, sem_ga)
            _scale_rows(rows_a, ca)
            dsa = _fire_s(rows_a, ca, sem_sa)

            _wait_g(rows_b, sem_gb)
            _scale_rows(rows_b, ca + 1)
            dsb = _fire_s(rows_b, ca + 1, sem_sb)

            dsa.wait()
            _fire_g(rows_a, ca + 3, sem_ga)

            _wait_g(rows_c, sem_gc)
            _scale_rows(rows_c, ca + 2)
            dsc = _fire_s(rows_c, ca + 2, sem_sc)

            dsb.wait()

            @pl.when(t < CPS // 3 - 1)
            def _():
                _fire_g(rows_b, ca + 4, sem_gb)

            dsc.wait()

            @pl.when(t < CPS // 3 - 1)
            def _():
                _fire_g(rows_c, ca + 5, sem_gc)
            return 0
        lax.fori_loop(0, CPS // 3, _body, 0)

        ct = CPS - 1
        _wait_g(rows_a, sem_ga)
        _scale_rows(rows_a, ct)
        _fire_s(rows_a, ct, sem_sa)
        _wait_s(rows_a, sem_sa)
        for i in range(CPS):
            pltpu.make_async_copy(
                w_g.at[0], p_sh.at[src_g.at[0]], sem_q).wait()
        return 0
    lax.fori_loop(0, NSLAB, _slab, 0)

    plsc.subcore_barrier()

    pltpu.sync_copy(b_sh.at[pl.ds(r0, NPT)], b_sl)

    def _out_block(rb, _):
        base = rb * CK

        @pl.when(r0 + base < N)
        def _():
            pltpu.sync_copy(y_sh.at[pl.ds(r0 + base, CK), :], rows_a)

        @pl.when(r0 + base >= N)
        def _():
            def _z(j, _2):
                for c in range(D // 16):
                    rows_a[j, pl.ds(c * 16, 16)] = jnp.zeros(
                        (16,), jnp.float32)
                return 0
            lax.fori_loop(0, CK, _z, 0)

        def _brow(g, _):
            bv16 = b_sl[pl.ds(base + g * 16, 16)]
            for rr in range(16):
                r = g * 16 + rr
                bv = bv16[rr]
                for c in range(D // 16):
                    sl = pl.ds(c * 16, 16)
                    rows_a[r, sl] = rows_a[r, sl] * bv
            return 0
        lax.fori_loop(0, CK // 16, _brow, 0)
        pltpu.sync_copy(rows_a, ys_hbm.at[cid, pl.ds(r0 + base, CK), :])
        return 0
    lax.fori_loop(0, RB, _out_block, 0)

    pltpu.sync_copy(p_sh.at[pl.ds(r0, NPT)], tmp_vm)

    pltpu.sync_copy(a_sh.at[pl.ds(r0, NPT)], b_sl)

    def _g(i, _):
        sl = pl.ds(i * 16, 16)
        tmp_vm[sl] = tmp_vm[sl] * b_sl[sl]
        return 0
    lax.fori_loop(0, NPT // 16, _g, 0)
    pltpu.sync_copy(tmp_vm, g_hbm.at[cid, pl.ds(r0, NPT)])


_sc_call = functools.partial(
    pl.kernel,
    out_type=(
        jax.ShapeDtypeStruct((NC, NP, D), jnp.float32),
        jax.ShapeDtypeStruct((NC, NP), jnp.float32),
    ),
    mesh=plsc.VectorSubcoreMesh(core_axis_name="c", subcore_axis_name="s"),
    scratch_types=[
        pltpu.VMEM((CPS, CK), jnp.int32),
        pltpu.VMEM((CPS, CK), jnp.int32),
        pltpu.VMEM((CPS, CK), jnp.float32),
        pltpu.VMEM((CPS, CK), jnp.float32),
        pltpu.VMEM((CK, D), jnp.float32),
        pltpu.VMEM((CK, D), jnp.float32),
        pltpu.VMEM((CK, D), jnp.float32),
        pltpu.VMEM((CK,), jnp.float32),
        pltpu.VMEM((NPT,), jnp.float32),
        pltpu.VMEM((NPT,), jnp.float32),
        pltpu.VMEM_SHARED((N, D), jnp.float32),
        pltpu.VMEM_SHARED((NP,), jnp.float32),
        pltpu.VMEM_SHARED((NP,), jnp.float32),
        pltpu.VMEM_SHARED((NP,), jnp.float32),
        pltpu.SemaphoreType.DMA,
        pltpu.SemaphoreType.DMA,
        pltpu.SemaphoreType.DMA,
        pltpu.SemaphoreType.DMA,
        pltpu.SemaphoreType.DMA,
        pltpu.SemaphoreType.DMA,
        pltpu.SemaphoreType.DMA,
    ],
)(_sc_body)


BLK = 256
NBLK = NP // BLK


def _tc_body(ysf, ysl, gf, gl, w0f, w0l, b0f, b0l, w1f, w1l, b1f, b1l,
             out, acc):
    i = pl.program_id(0)

    @pl.when(i == 0)
    def _():
        acc[...] = jnp.zeros((2, D), jnp.float32)

    h = jnp.maximum(
        jnp.dot(ysf[...], w0f[...], preferred_element_type=jnp.float32)
        + jnp.dot(ysl[...], w0l[...], preferred_element_type=jnp.float32)
        + b0f[...] + b0l[...],
        0.0,
    )
    acc[0:1, :] += jnp.dot(gf[0], h, preferred_element_type=jnp.float32)
    acc[1:2, :] += jnp.dot(gl[0], h, preferred_element_type=jnp.float32)

    @pl.when(i == NBLK - 1)
    def _():
        out[...] = (
            jnp.dot(acc[0:1, :] * (1.0 / N), w1f[...],
                    preferred_element_type=jnp.float32)
            + jnp.dot(acc[1:2, :] * (1.0 / N), w1l[...],
                      preferred_element_type=jnp.float32)
            + b1f[...] + b1l[...]
        )


_tc_call = pl.pallas_call(
    _tc_body,
    grid=(NBLK,),
    in_specs=[
        pl.BlockSpec((BLK, D), lambda i: (i, 0)),
        pl.BlockSpec((BLK, D), lambda i: (i, 0)),
        pl.BlockSpec((1, 1, BLK), lambda i: (i, 0, 0)),
        pl.BlockSpec((1, 1, BLK), lambda i: (i, 0, 0)),
        pl.BlockSpec((D, D), lambda i: (0, 0)),
        pl.BlockSpec((D, D), lambda i: (0, 0)),
        pl.BlockSpec((1, D), lambda i: (0, 0)),
        pl.BlockSpec((1, D), lambda i: (0, 0)),
        pl.BlockSpec((D, D), lambda i: (0, 0)),
        pl.BlockSpec((D, D), lambda i: (0, 0)),
        pl.BlockSpec((1, D), lambda i: (0, 0)),
        pl.BlockSpec((1, D), lambda i: (0, 0)),
    ],
    out_specs=pl.BlockSpec((1, D), lambda i: (0, 0)),
    out_shape=jax.ShapeDtypeStruct((1, D), jnp.float32),
    scratch_shapes=[pltpu.VMEM((2, D), jnp.float32)],
)


def kernel(x, edge_index_follows, edge_weight_follows, edge_index_likes,
           edge_weight_likes, W0_follows, b0_follows, W0_likes, b0_likes,
           W1_follows, b1_follows, W1_likes, b1_likes):
    shp = (NS, NSLAB, CPS, CK)
    ys, g = _sc_call(
        x,
        edge_index_follows[0].reshape(shp), edge_index_follows[1].reshape(shp),
        edge_weight_follows.reshape(shp),
        edge_index_likes[0].reshape(shp), edge_index_likes[1].reshape(shp),
        edge_weight_likes.reshape(shp),
    )

    out = _tc_call(
        ys[0], ys[1],
        g[0].reshape(NBLK, 1, BLK), g[1].reshape(NBLK, 1, BLK),
        W0_follows, W0_likes,
        b0_follows.reshape(1, D), b0_likes.reshape(1, D),
        W1_follows, W1_likes,
        b1_follows.reshape(1, D), b1_likes.reshape(1, D),
    )
    return out


# speedup vs baseline: 13.6770x; 1.0669x over previous
import functools
import jax
import jax.numpy as jnp
from jax import lax
from jax.experimental import pallas as pl
from jax.experimental.pallas import tpu as pltpu
from jax.experimental.pallas import tpu_sc as plsc

N = 10000
D = 128
E = 160000
NC = 2
NS = 16
NP = 10240
NPT = NP // NS
EPT = E // NS
CK = 80
NCH = EPT // CK
CPS = 25
NSLAB = NCH // CPS
RB = NPT // CK
def _rsqrt16(v):
    s = (v + 1.0) * 0.5
    for _ in range(16):
        s = 0.5 * (s + v / s)
    return 1.0 / s


def _sc_body(x_hbm, srcf_hbm, dstf_hbm, wf_hbm, srcl_hbm, dstl_hbm,
             wl_hbm, ys_hbm, g_hbm,
             src_g, dst_g, w_g, ga_g, rows_a, rows_b, rows_c,
             ones_vm, tmp_vm, b_sl,
             y_sh, p_sh, a_sh, b_sh,
             sem_ga, sem_gb, sem_gc, sem_sa, sem_sb, sem_sc, sem_q):

    def _load_slab(s_, want_w):
        cid_ = lax.axis_index("c")
        sid_ = lax.axis_index("s")

        @pl.when(cid_ == 0)
        def _():
            pltpu.sync_copy(srcf_hbm.at[sid_, s_], src_g)
            pltpu.sync_copy(dstf_hbm.at[sid_, s_], dst_g)
            if want_w:
                pltpu.sync_copy(wf_hbm.at[sid_, s_], w_g)

        @pl.when(cid_ == 1)
        def _():
            pltpu.sync_copy(srcl_hbm.at[sid_, s_], src_g)
            pltpu.sync_copy(dstl_hbm.at[sid_, s_], dst_g)
            if want_w:
                pltpu.sync_copy(wl_hbm.at[sid_, s_], w_g)
    cid = lax.axis_index("c")
    sid = lax.axis_index("s")
    r0 = sid * NPT

    z16 = jnp.zeros((16,), jnp.float32)

    def _zrow(j, _):
        for c in range(D // 16):
            rows_a[j, pl.ds(c * 16, 16)] = z16
        return 0
    lax.fori_loop(0, CK, _zrow, 0)

    def _ztmp(i, _):
        tmp_vm[pl.ds(i * 16, 16)] = z16
        return 0
    lax.fori_loop(0, NPT // 16, _ztmp, 0)
    for i in range(CK // 16):
        ones_vm[pl.ds(i * 16, 16)] = jnp.full((16,), 1.0, jnp.float32)

    for rb in range(RB):
        @pl.when(r0 + rb * CK < N)
        def _():
            pltpu.sync_copy(rows_a, y_sh.at[pl.ds(r0 + rb * CK, CK), :])
    pltpu.sync_copy(tmp_vm, a_sh.at[pl.ds(r0, NPT)])
    pltpu.sync_copy(tmp_vm, b_sh.at[pl.ds(r0, NPT)])
    pltpu.sync_copy(tmp_vm, p_sh.at[pl.ds(r0, NPT)])

    plsc.subcore_barrier()

    def _deg(s_, _):
        _load_slab(s_, False)
        descs = []
        for i in range(CPS):
            descs.append(pltpu.async_copy(
                ones_vm, a_sh.at[src_g.at[i]], sem_q, add=True))
            descs.append(pltpu.async_copy(
                ones_vm, b_sh.at[dst_g.at[i]], sem_q, add=True))
        for d in descs:
            d.wait()
        return 0
    lax.fori_loop(0, NSLAB, _deg, 0)

    plsc.subcore_barrier()

    pltpu.sync_copy(a_sh.at[pl.ds(r0, NPT)], tmp_vm)

    def _ra(i, _):
        v = jnp.maximum(tmp_vm[pl.ds(i * 16, 16)], 1.0)
        tmp_vm[pl.ds(i * 16, 16)] = _rsqrt16(v)
        return 0
    lax.fori_loop(0, NPT // 16, _ra, 0)
    pltpu.sync_copy(tmp_vm, a_sh.at[pl.ds(r0, NPT)])

    pltpu.sync_copy(b_sh.at[pl.ds(r0, NPT)], tmp_vm)
    lax.fori_loop(0, NPT // 16, _ra, 0)
    pltpu.sync_copy(tmp_vm, b_sh.at[pl.ds(r0, NPT)])

    plsc.subcore_barrier()

    def _scale_rows(rows_ref, ci):
        def _sg(g, _):
            mv16 = ga_g[ci, pl.ds(g * 16, 16)]
            for rr in range(16):
                r = g * 16 + rr
                mv = mv16[rr]
                for c in range(D // 16):
                    sl = pl.ds(c * 16, 16)
                    rows_ref[r, sl] = rows_ref[r, sl] * mv
            return 0
        lax.fori_loop(0, CK // 16, _sg, 0)

    def _fire_g(slot_ref, ci, sem):
        return pltpu.async_copy(x_hbm.at[src_g.at[ci]], slot_ref, sem)

    def _wait_g(slot_ref, sem):
        pltpu.make_async_copy(x_hbm.at[src_g.at[0]], slot_ref, sem).wait()

    def _fire_s(slot_ref, ci, sem):
        return pltpu.async_copy(
            slot_ref, y_sh.at[dst_g.at[ci]], sem, add=True)

    def _wait_s(slot_ref, sem):
        pltpu.make_async_copy(
            slot_ref, y_sh.at[dst_g.at[0]], sem).wait()

    def _slab(s_, _):
        _load_slab(s_, True)

        descs = []
        for i in range(CPS):
            descs.append(pltpu.async_copy(
                a_sh.at[src_g.at[i]], ga_g.at[i], sem_q))
            descs.append(pltpu.async_copy(
                b_sh.at[dst_g.at[i]], rows_a.at[i, pl.ds(0, CK)], sem_q))
        for d in descs:
            d.wait()

        def _mq(i, _):
            for k in range(CK // 16):
                sl = pl.ds(k * 16, 16)
                wv = w_g[i, sl]
                ga_g[i, sl] = wv * ga_g[i, sl]
                w_g[i, sl] = wv * rows_a[i, sl]
            return 0
        lax.fori_loop(0, CPS, _mq, 0)

        for i in range(CPS):
            pltpu.async_copy(w_g.at[i], p_sh.at[src_g.at[i]], sem_q,
                             add=True)

        _fire_g(rows_a, 0, sem_ga)
        _fire_g(rows_b, 1, sem_gb)
        _fire_g(rows_c, 2, sem_gc)

        def _body(t, _):
            ca = 3 * t
            _wait_g(rows_a, sem_ga)
            _scale_rows(rows_a, ca)
            dsa = _fire_s(rows_a, ca, sem_sa)

            _wait_g(rows_b, sem_gb)
            _scale_rows(rows_b, ca + 1)
            dsb = _fire_s(rows_b, ca + 1, sem_sb)

            dsa.wait()
            _fire_g(rows_a, ca + 3, sem_ga)

            _wait_g(rows_c, sem_gc)
            _scale_rows(rows_c, ca + 2)
            dsc = _fire_s(rows_c, ca + 2, sem_sc)

            dsb.wait()

            @pl.when(t < CPS // 3 - 1)
            def _():
                _fire_g(rows_b, ca + 4, sem_gb)

            dsc.wait()

            @pl.when(t < CPS // 3 - 1)
            def _():
                _fire_g(rows_c, ca + 5, sem_gc)
            return 0
        lax.fori_loop(0, CPS // 3, _body, 0)

        ct = CPS - 1
        _wait_g(rows_a, sem_ga)
        _scale_rows(rows_a, ct)
        _fire_s(rows_a, ct, sem_sa)
        _wait_s(rows_a, sem_sa)
        for i in range(CPS):
            pltpu.make_async_copy(
                w_g.at[0], p_sh.at[src_g.at[0]], sem_q).wait()
        return 0
    lax.fori_loop(0, NSLAB, _slab, 0)

    plsc.subcore_barrier()

    pltpu.sync_copy(b_sh.at[pl.ds(r0, NPT)], b_sl)

    def _out_block(rb, _):
        base = rb * CK

        @pl.when(r0 + base < N)
        def _():
            pltpu.sync_copy(y_sh.at[pl.ds(r0 + base, CK), :], rows_a)

        @pl.when(r0 + base >= N)
        def _():
            def _z(j, _2):
                for c in range(D // 16):
                    rows_a[j, pl.ds(c * 16, 16)] = jnp.zeros(
                        (16,), jnp.float32)
                return 0
            lax.fori_loop(0, CK, _z, 0)

        def _brow(g, _):
            bv16 = b_sl[pl.ds(base + g * 16, 16)]
            for rr in range(16):
                r = g * 16 + rr
                bv = bv16[rr]
                for c in range(D // 16):
                    sl = pl.ds(c * 16, 16)
                    rows_a[r, sl] = rows_a[r, sl] * bv
            return 0
        lax.fori_loop(0, CK // 16, _brow, 0)
        pltpu.sync_copy(rows_a, ys_hbm.at[cid, pl.ds(r0 + base, CK), :])
        return 0
    lax.fori_loop(0, RB, _out_block, 0)

    pltpu.sync_copy(p_sh.at[pl.ds(r0, NPT)], tmp_vm)

    pltpu.sync_copy(a_sh.at[pl.ds(r0, NPT)], b_sl)

    def _g(i, _):
        sl = pl.ds(i * 16, 16)
        tmp_vm[sl] = tmp_vm[sl] * b_sl[sl]
        return 0
    lax.fori_loop(0, NPT // 16, _g, 0)
    pltpu.sync_copy(tmp_vm, g_hbm.at[cid, pl.ds(r0, NPT)])


_sc_call = functools.partial(
    pl.kernel,
    out_type=(
        jax.ShapeDtypeStruct((NC, NP, D), jnp.float32),
        jax.ShapeDtypeStruct((NC, NP), jnp.float32),
    ),
    mesh=plsc.VectorSubcoreMesh(core_axis_name="c", subcore_axis_name="s"),
    scratch_types=[
        pltpu.VMEM((CPS, CK), jnp.int32),
        pltpu.VMEM((CPS, CK), jnp.int32),
        pltpu.VMEM((CPS, CK), jnp.float32),
        pltpu.VMEM((CPS, CK), jnp.float32),
        pltpu.VMEM((CK, D), jnp.float32),
        pltpu.VMEM((CK, D), jnp.float32),
        pltpu.VMEM((CK, D), jnp.float32),
        pltpu.VMEM((CK,), jnp.float32),
        pltpu.VMEM((NPT,), jnp.float32),
        pltpu.VMEM((NPT,), jnp.float32),
        pltpu.VMEM_SHARED((N, D), jnp.float32),
        pltpu.VMEM_SHARED((NP,), jnp.float32),
        pltpu.VMEM_SHARED((NP,), jnp.float32),
        pltpu.VMEM_SHARED((NP,), jnp.float32),
        pltpu.SemaphoreType.DMA,
        pltpu.SemaphoreType.DMA,
        pltpu.SemaphoreType.DMA,
        pltpu.SemaphoreType.DMA,
        pltpu.SemaphoreType.DMA,
        pltpu.SemaphoreType.DMA,
        pltpu.SemaphoreType.DMA,
    ],
)(_sc_body)


BLK = 1024
NBLK = NP // BLK


def _tc_body(ysf, ysl, gf, gl, w0f, w0l, b0f, b0l, w1f, w1l, b1f, b1l,
             out, acc):
    i = pl.program_id(0)

    @pl.when(i == 0)
    def _():
        acc[...] = jnp.zeros((2, D), jnp.float32)

    h = jnp.maximum(
        jnp.dot(ysf[...], w0f[...], preferred_element_type=jnp.float32)
        + jnp.dot(ysl[...], w0l[...], preferred_element_type=jnp.float32)
        + b0f[...] + b0l[...],
        0.0,
    )
    acc[0:1, :] += jnp.dot(gf[0], h, preferred_element_type=jnp.float32)
    acc[1:2, :] += jnp.dot(gl[0], h, preferred_element_type=jnp.float32)

    @pl.when(i == NBLK - 1)
    def _():
        out[...] = (
            jnp.dot(acc[0:1, :] * (1.0 / N), w1f[...],
                    preferred_element_type=jnp.float32)
            + jnp.dot(acc[1:2, :] * (1.0 / N), w1l[...],
                      preferred_element_type=jnp.float32)
            + b1f[...] + b1l[...]
        )


_tc_call = pl.pallas_call(
    _tc_body,
    grid=(NBLK,),
    in_specs=[
        pl.BlockSpec((BLK, D), lambda i: (i, 0)),
        pl.BlockSpec((BLK, D), lambda i: (i, 0)),
        pl.BlockSpec((1, 1, BLK), lambda i: (i, 0, 0)),
        pl.BlockSpec((1, 1, BLK), lambda i: (i, 0, 0)),
        pl.BlockSpec((D, D), lambda i: (0, 0)),
        pl.BlockSpec((D, D), lambda i: (0, 0)),
        pl.BlockSpec((1, D), lambda i: (0, 0)),
        pl.BlockSpec((1, D), lambda i: (0, 0)),
        pl.BlockSpec((D, D), lambda i: (0, 0)),
        pl.BlockSpec((D, D), lambda i: (0, 0)),
        pl.BlockSpec((1, D), lambda i: (0, 0)),
        pl.BlockSpec((1, D), lambda i: (0, 0)),
    ],
    out_specs=pl.BlockSpec((1, D), lambda i: (0, 0)),
    out_shape=jax.ShapeDtypeStruct((1, D), jnp.float32),
    scratch_shapes=[pltpu.VMEM((2, D), jnp.float32)],
)


def kernel(x, edge_index_follows, edge_weight_follows, edge_index_likes,
           edge_weight_likes, W0_follows, b0_follows, W0_likes, b0_likes,
           W1_follows, b1_follows, W1_likes, b1_likes):
    shp = (NS, NSLAB, CPS, CK)
    ys, g = _sc_call(
        x,
        edge_index_follows[0].reshape(shp), edge_index_follows[1].reshape(shp),
        edge_weight_follows.reshape(shp),
        edge_index_likes[0].reshape(shp), edge_index_likes[1].reshape(shp),
        edge_weight_likes.reshape(shp),
    )

    out = _tc_call(
        ys[0], ys[1],
        g[0].reshape(NBLK, 1, BLK), g[1].reshape(NBLK, 1, BLK),
        W0_follows, W0_likes,
        b0_follows.reshape(1, D), b0_likes.reshape(1, D),
        W1_follows, W1_likes,
        b1_follows.reshape(1, D), b1_likes.reshape(1, D),
    )
    return out


# speedup vs baseline: 14.3624x; 1.0501x over previous
import functools
import jax
import jax.numpy as jnp
from jax import lax
from jax.experimental import pallas as pl
from jax.experimental.pallas import tpu as pltpu
from jax.experimental.pallas import tpu_sc as plsc

N = 10000
D = 128
E = 160000
NC = 2
NS = 16
NP = 10240
NPT = NP // NS
EPT = E // NS
CK = 80
NCH = EPT // CK
CPS = 25
NSLAB = NCH // CPS
RB = NPT // CK
def _rsqrt16(v):
    s = (v + 1.0) * 0.5
    for _ in range(16):
        s = 0.5 * (s + v / s)
    return 1.0 / s


def _sc_body(x_hbm, srcf_hbm, dstf_hbm, wf_hbm, srcl_hbm, dstl_hbm,
             wl_hbm, ys_hbm, g_hbm,
             src_g, dst_g, w_g, ga_g, rows_a, rows_b, rows_c,
             ones_vm, tmp_vm, b_sl,
             y_sh, p_sh, a_sh, b_sh,
             sem_ga, sem_gb, sem_gc, sem_sa, sem_sb, sem_sc, sem_q):

    def _load_slab(s_, want_w):
        cid_ = lax.axis_index("c")
        sid_ = lax.axis_index("s")

        @pl.when(cid_ == 0)
        def _():
            pltpu.sync_copy(srcf_hbm.at[sid_, s_], src_g)
            pltpu.sync_copy(dstf_hbm.at[sid_, s_], dst_g)
            if want_w:
                pltpu.sync_copy(wf_hbm.at[sid_, s_], w_g)

        @pl.when(cid_ == 1)
        def _():
            pltpu.sync_copy(srcl_hbm.at[sid_, s_], src_g)
            pltpu.sync_copy(dstl_hbm.at[sid_, s_], dst_g)
            if want_w:
                pltpu.sync_copy(wl_hbm.at[sid_, s_], w_g)
    cid = lax.axis_index("c")
    sid = lax.axis_index("s")
    r0 = sid * NPT

    z16 = jnp.zeros((16,), jnp.float32)

    def _zrow(j, _):
        for c in range(D // 16):
            rows_a[j, pl.ds(c * 16, 16)] = z16
        return 0
    lax.fori_loop(0, CK, _zrow, 0)

    def _ztmp(i, _):
        tmp_vm[pl.ds(i * 16, 16)] = z16
        return 0
    lax.fori_loop(0, NPT // 16, _ztmp, 0)
    for i in range(CK // 16):
        ones_vm[pl.ds(i * 16, 16)] = jnp.full((16,), 1.0, jnp.float32)

    for rb in range(RB):
        @pl.when(r0 + rb * CK < N)
        def _():
            pltpu.sync_copy(rows_a, y_sh.at[pl.ds(r0 + rb * CK, CK), :])
    pltpu.sync_copy(tmp_vm, a_sh.at[pl.ds(r0, NPT)])
    pltpu.sync_copy(tmp_vm, b_sh.at[pl.ds(r0, NPT)])
    pltpu.sync_copy(tmp_vm, p_sh.at[pl.ds(r0, NPT)])

    plsc.subcore_barrier()

    def _deg(s_, _):
        _load_slab(s_, False)
        descs = []
        for i in range(CPS):
            descs.append(pltpu.async_copy(
                ones_vm, a_sh.at[src_g.at[i]], sem_q, add=True))
            descs.append(pltpu.async_copy(
                ones_vm, b_sh.at[dst_g.at[i]], sem_q, add=True))
        for d in descs:
            d.wait()
        return 0
    lax.fori_loop(0, NSLAB, _deg, 0)

    plsc.subcore_barrier()

    pltpu.sync_copy(a_sh.at[pl.ds(r0, NPT)], tmp_vm)

    def _ra(i, _):
        v = jnp.maximum(tmp_vm[pl.ds(i * 16, 16)], 1.0)
        tmp_vm[pl.ds(i * 16, 16)] = _rsqrt16(v)
        return 0
    lax.fori_loop(0, NPT // 16, _ra, 0)
    pltpu.sync_copy(tmp_vm, a_sh.at[pl.ds(r0, NPT)])

    pltpu.sync_copy(b_sh.at[pl.ds(r0, NPT)], tmp_vm)
    lax.fori_loop(0, NPT // 16, _ra, 0)
    pltpu.sync_copy(tmp_vm, b_sh.at[pl.ds(r0, NPT)])

    plsc.subcore_barrier()

    def _scale_rows(rows_ref, ci):
        def _sg(g, _):
            mv16 = ga_g[ci, pl.ds(g * 16, 16)]
            for rr in range(16):
                r = g * 16 + rr
                mv = mv16[rr]
                for c in range(D // 16):
                    sl = pl.ds(c * 16, 16)
                    rows_ref[r, sl] = rows_ref[r, sl] * mv
            return 0
        lax.fori_loop(0, CK // 16, _sg, 0)

    def _fire_g(slot_ref, ci, sem):
        return pltpu.async_copy(x_hbm.at[src_g.at[ci]], slot_ref, sem)

    def _wait_g(slot_ref, sem):
        pltpu.make_async_copy(x_hbm.at[src_g.at[0]], slot_ref, sem).wait()

    def _fire_s(slot_ref, ci, sem):
        return pltpu.async_copy(
            slot_ref, y_sh.at[dst_g.at[ci]], sem, add=True)

    def _wait_s(slot_ref, sem):
        pltpu.make_async_copy(
            slot_ref, y_sh.at[dst_g.at[0]], sem).wait()

    def _slab(s_, _):
        _load_slab(s_, True)

        _fire_g(rows_a, 0, sem_ga)
        _fire_g(rows_b, 1, sem_gb)

        descs = []
        for i in range(CPS):
            descs.append(pltpu.async_copy(
                a_sh.at[src_g.at[i]], ga_g.at[i], sem_q))
            descs.append(pltpu.async_copy(
                b_sh.at[dst_g.at[i]], rows_c.at[i, pl.ds(0, CK)], sem_q))
        for d in descs:
            d.wait()

        def _mq(i, _):
            for k in range(CK // 16):
                sl = pl.ds(k * 16, 16)
                wv = w_g[i, sl]
                ga_g[i, sl] = wv * ga_g[i, sl]
                w_g[i, sl] = wv * rows_c[i, sl]
            return 0
        lax.fori_loop(0, CPS, _mq, 0)

        for i in range(CPS):
            pltpu.async_copy(w_g.at[i], p_sh.at[src_g.at[i]], sem_q,
                             add=True)
        _fire_g(rows_c, 2, sem_gc)

        def _body(t, _):
            ca = 3 * t
            _wait_g(rows_a, sem_ga)
            _scale_rows(rows_a, ca)
            dsa = _fire_s(rows_a, ca, sem_sa)

            _wait_g(rows_b, sem_gb)
            _scale_rows(rows_b, ca + 1)
            dsb = _fire_s(rows_b, ca + 1, sem_sb)

            dsa.wait()
            _fire_g(rows_a, ca + 3, sem_ga)

            _wait_g(rows_c, sem_gc)
            _scale_rows(rows_c, ca + 2)
            dsc = _fire_s(rows_c, ca + 2, sem_sc)

            dsb.wait()

            @pl.when(t < CPS // 3 - 1)
            def _():
                _fire_g(rows_b, ca + 4, sem_gb)

            dsc.wait()

            @pl.when(t < CPS // 3 - 1)
            def _():
                _fire_g(rows_c, ca + 5, sem_gc)
            return 0
        lax.fori_loop(0, CPS // 3, _body, 0)

        ct = CPS - 1
        _wait_g(rows_a, sem_ga)
        _scale_rows(rows_a, ct)
        _fire_s(rows_a, ct, sem_sa)
        _wait_s(rows_a, sem_sa)
        for i in range(CPS):
            pltpu.make_async_copy(
                w_g.at[0], p_sh.at[src_g.at[0]], sem_q).wait()
        return 0
    lax.fori_loop(0, NSLAB, _slab, 0)

    plsc.subcore_barrier()

    pltpu.sync_copy(b_sh.at[pl.ds(r0, NPT)], b_sl)

    def _out_block(rb, _):
        base = rb * CK

        @pl.when(r0 + base < N)
        def _():
            pltpu.sync_copy(y_sh.at[pl.ds(r0 + base, CK), :], rows_a)

        @pl.when(r0 + base >= N)
        def _():
            def _z(j, _2):
                for c in range(D // 16):
                    rows_a[j, pl.ds(c * 16, 16)] = jnp.zeros(
                        (16,), jnp.float32)
                return 0
            lax.fori_loop(0, CK, _z, 0)

        def _brow(g, _):
            bv16 = b_sl[pl.ds(base + g * 16, 16)]
            for rr in range(16):
                r = g * 16 + rr
                bv = bv16[rr]
                for c in range(D // 16):
                    sl = pl.ds(c * 16, 16)
                    rows_a[r, sl] = rows_a[r, sl] * bv
            return 0
        lax.fori_loop(0, CK // 16, _brow, 0)
        pltpu.sync_copy(rows_a, ys_hbm.at[cid, pl.ds(r0 + base, CK), :])
        return 0
    lax.fori_loop(0, RB, _out_block, 0)

    pltpu.sync_copy(p_sh.at[pl.ds(r0, NPT)], tmp_vm)

    pltpu.sync_copy(a_sh.at[pl.ds(r0, NPT)], b_sl)

    def _g(i, _):
        sl = pl.ds(i * 16, 16)
        tmp_vm[sl] = tmp_vm[sl] * b_sl[sl]
        return 0
    lax.fori_loop(0, NPT // 16, _g, 0)
    pltpu.sync_copy(tmp_vm, g_hbm.at[cid, pl.ds(r0, NPT)])


_sc_call = functools.partial(
    pl.kernel,
    out_type=(
        jax.ShapeDtypeStruct((NC, NP, D), jnp.float32),
        jax.ShapeDtypeStruct((NC, NP), jnp.float32),
    ),
    mesh=plsc.VectorSubcoreMesh(core_axis_name="c", subcore_axis_name="s"),
    scratch_types=[
        pltpu.VMEM((CPS, CK), jnp.int32),
        pltpu.VMEM((CPS, CK), jnp.int32),
        pltpu.VMEM((CPS, CK), jnp.float32),
        pltpu.VMEM((CPS, CK), jnp.float32),
        pltpu.VMEM((CK, D), jnp.float32),
        pltpu.VMEM((CK, D), jnp.float32),
        pltpu.VMEM((CK, D), jnp.float32),
        pltpu.VMEM((CK,), jnp.float32),
        pltpu.VMEM((NPT,), jnp.float32),
        pltpu.VMEM((NPT,), jnp.float32),
        pltpu.VMEM_SHARED((N, D), jnp.float32),
        pltpu.VMEM_SHARED((NP,), jnp.float32),
        pltpu.VMEM_SHARED((NP,), jnp.float32),
        pltpu.VMEM_SHARED((NP,), jnp.float32),
        pltpu.SemaphoreType.DMA,
        pltpu.SemaphoreType.DMA,
        pltpu.SemaphoreType.DMA,
        pltpu.SemaphoreType.DMA,
        pltpu.SemaphoreType.DMA,
        pltpu.SemaphoreType.DMA,
        pltpu.SemaphoreType.DMA,
    ],
)(_sc_body)


BLK = 1024
NBLK = NP // BLK


def _tc_body(ysf, ysl, gf, gl, w0f, w0l, b0f, b0l, w1f, w1l, b1f, b1l,
             out, acc):
    i = pl.program_id(0)

    @pl.when(i == 0)
    def _():
        acc[...] = jnp.zeros((2, D), jnp.float32)

    h = jnp.maximum(
        jnp.dot(ysf[...], w0f[...], preferred_element_type=jnp.float32)
        + jnp.dot(ysl[...], w0l[...], preferred_element_type=jnp.float32)
        + b0f[...] + b0l[...],
        0.0,
    )
    acc[0:1, :] += jnp.dot(gf[0], h, preferred_element_type=jnp.float32)
    acc[1:2, :] += jnp.dot(gl[0], h, preferred_element_type=jnp.float32)

    @pl.when(i == NBLK - 1)
    def _():
        out[...] = (
            jnp.dot(acc[0:1, :] * (1.0 / N), w1f[...],
                    preferred_element_type=jnp.float32)
            + jnp.dot(acc[1:2, :] * (1.0 / N), w1l[...],
                      preferred_element_type=jnp.float32)
            + b1f[...] + b1l[...]
        )


_tc_call = pl.pallas_call(
    _tc_body,
    grid=(NBLK,),
    in_specs=[
        pl.BlockSpec((BLK, D), lambda i: (i, 0)),
        pl.BlockSpec((BLK, D), lambda i: (i, 0)),
        pl.BlockSpec((1, 1, BLK), lambda i: (i, 0, 0)),
        pl.BlockSpec((1, 1, BLK), lambda i: (i, 0, 0)),
        pl.BlockSpec((D, D), lambda i: (0, 0)),
        pl.BlockSpec((D, D), lambda i: (0, 0)),
        pl.BlockSpec((1, D), lambda i: (0, 0)),
        pl.BlockSpec((1, D), lambda i: (0, 0)),
        pl.BlockSpec((D, D), lambda i: (0, 0)),
        pl.BlockSpec((D, D), lambda i: (0, 0)),
        pl.BlockSpec((1, D), lambda i: (0, 0)),
        pl.BlockSpec((1, D), lambda i: (0, 0)),
    ],
    out_specs=pl.BlockSpec((1, D), lambda i: (0, 0)),
    out_shape=jax.ShapeDtypeStruct((1, D), jnp.float32),
    scratch_shapes=[pltpu.VMEM((2, D), jnp.float32)],
)


def kernel(x, edge_index_follows, edge_weight_follows, edge_index_likes,
           edge_weight_likes, W0_follows, b0_follows, W0_likes, b0_likes,
           W1_follows, b1_follows, W1_likes, b1_likes):
    shp = (NS, NSLAB, CPS, CK)
    ys, g = _sc_call(
        x,
        edge_index_follows[0].reshape(shp), edge_index_follows[1].reshape(shp),
        edge_weight_follows.reshape(shp),
        edge_index_likes[0].reshape(shp), edge_index_likes[1].reshape(shp),
        edge_weight_likes.reshape(shp),
    )

    out = _tc_call(
        ys[0], ys[1],
        g[0].reshape(NBLK, 1, BLK), g[1].reshape(NBLK, 1, BLK),
        W0_follows, W0_likes,
        b0_follows.reshape(1, D), b0_likes.reshape(1, D),
        W1_follows, W1_likes,
        b1_follows.reshape(1, D), b1_likes.reshape(1, D),
    )
    return out


# speedup vs baseline: 14.3918x; 1.0020x over previous
import functools
import jax
import jax.numpy as jnp
from jax import lax
from jax.experimental import pallas as pl
from jax.experimental.pallas import tpu as pltpu
from jax.experimental.pallas import tpu_sc as plsc

N = 10000
D = 128
E = 160000
NC = 2
NS = 16
NP = 10240
NPT = NP // NS
EPT = E // NS
CK = 80
NCH = EPT // CK
CPS = 25
NSLAB = NCH // CPS
RB = NPT // CK
def _rsqrt16(v):
    s = (v + 1.0) * 0.5
    for _ in range(16):
        s = 0.5 * (s + v / s)
    return 1.0 / s


def _sc_body(x_hbm, srcf_hbm, dstf_hbm, wf_hbm, srcl_hbm, dstl_hbm,
             wl_hbm, ys_hbm, g_hbm,
             src_g, dst_g, w_g, ga_g, rows_a, rows_b, rows_c,
             ones_vm, tmp_vm, b_sl,
             y_sh, p_sh, a_sh, b_sh,
             sem_ga, sem_gb, sem_gc, sem_sa, sem_sb, sem_sc, sem_q):

    def _load_slab(s_, want_w):
        cid_ = lax.axis_index("c")
        sid_ = lax.axis_index("s")

        @pl.when(cid_ == 0)
        def _():
            pltpu.sync_copy(srcf_hbm.at[sid_, s_], src_g)
            pltpu.sync_copy(dstf_hbm.at[sid_, s_], dst_g)
            if want_w:
                pltpu.sync_copy(wf_hbm.at[sid_, s_], w_g)

        @pl.when(cid_ == 1)
        def _():
            pltpu.sync_copy(srcl_hbm.at[sid_, s_], src_g)
            pltpu.sync_copy(dstl_hbm.at[sid_, s_], dst_g)
            if want_w:
                pltpu.sync_copy(wl_hbm.at[sid_, s_], w_g)
    cid = lax.axis_index("c")
    sid = lax.axis_index("s")
    r0 = sid * NPT

    z16 = jnp.zeros((16,), jnp.float32)

    def _zrow(j, _):
        for c in range(D // 16):
            rows_a[j, pl.ds(c * 16, 16)] = z16
        return 0
    lax.fori_loop(0, CK, _zrow, 0)

    def _ztmp(i, _):
        tmp_vm[pl.ds(i * 16, 16)] = z16
        return 0
    lax.fori_loop(0, NPT // 16, _ztmp, 0)
    for i in range(CK // 16):
        ones_vm[pl.ds(i * 16, 16)] = jnp.full((16,), 1.0, jnp.float32)

    pltpu.sync_copy(tmp_vm, a_sh.at[pl.ds(r0, NPT)])
    pltpu.sync_copy(tmp_vm, b_sh.at[pl.ds(r0, NPT)])
    pltpu.sync_copy(tmp_vm, p_sh.at[pl.ds(r0, NPT)])

    plsc.subcore_barrier()

    def _deg(s_, _):
        _load_slab(s_, False)
        descs = []
        for i in range(CPS):
            descs.append(pltpu.async_copy(
                ones_vm, a_sh.at[src_g.at[i]], sem_q, add=True))
            descs.append(pltpu.async_copy(
                ones_vm, b_sh.at[dst_g.at[i]], sem_q, add=True))
        for k in range(2):
            rb = 2 * s_ + k

            @pl.when((rb < RB) & (r0 + rb * CK < N))
            def _():
                pltpu.sync_copy(rows_a, y_sh.at[pl.ds(r0 + rb * CK, CK), :])
        for d in descs:
            d.wait()
        return 0
    lax.fori_loop(0, NSLAB, _deg, 0)

    plsc.subcore_barrier()

    pltpu.sync_copy(a_sh.at[pl.ds(r0, NPT)], tmp_vm)

    def _ra(i, _):
        v = jnp.maximum(tmp_vm[pl.ds(i * 16, 16)], 1.0)
        tmp_vm[pl.ds(i * 16, 16)] = _rsqrt16(v)
        return 0
    lax.fori_loop(0, NPT // 16, _ra, 0)
    pltpu.sync_copy(tmp_vm, a_sh.at[pl.ds(r0, NPT)])

    pltpu.sync_copy(b_sh.at[pl.ds(r0, NPT)], tmp_vm)
    lax.fori_loop(0, NPT // 16, _ra, 0)
    pltpu.sync_copy(tmp_vm, b_sh.at[pl.ds(r0, NPT)])

    plsc.subcore_barrier()

    def _scale_rows(rows_ref, ci):
        def _sg(g, _):
            mv16 = ga_g[ci, pl.ds(g * 16, 16)]
            for rr in range(16):
                r = g * 16 + rr
                mv = mv16[rr]
                for c in range(D // 16):
                    sl = pl.ds(c * 16, 16)
                    rows_ref[r, sl] = rows_ref[r, sl] * mv
            return 0
        lax.fori_loop(0, CK // 16, _sg, 0)

    def _fire_g(slot_ref, ci, sem):
        return pltpu.async_copy(x_hbm.at[src_g.at[ci]], slot_ref, sem)

    def _wait_g(slot_ref, sem):
        pltpu.make_async_copy(x_hbm.at[src_g.at[0]], slot_ref, sem).wait()

    def _fire_s(slot_ref, ci, sem):
        return pltpu.async_copy(
            slot_ref, y_sh.at[dst_g.at[ci]], sem, add=True)

    def _wait_s(slot_ref, sem):
        pltpu.make_async_copy(
            slot_ref, y_sh.at[dst_g.at[0]], sem).wait()

    def _slab(s_, _):
        _load_slab(s_, True)

        _fire_g(rows_a, 0, sem_ga)
        _fire_g(rows_b, 1, sem_gb)

        descs = []
        for i in range(CPS):
            descs.append(pltpu.async_copy(
                a_sh.at[src_g.at[i]], ga_g.at[i], sem_q))
            descs.append(pltpu.async_copy(
                b_sh.at[dst_g.at[i]], rows_c.at[i, pl.ds(0, CK)], sem_q))
        for d in descs:
            d.wait()

        def _mq(i, _):
            for k in range(CK // 16):
                sl = pl.ds(k * 16, 16)
                wv = w_g[i, sl]
                ga_g[i, sl] = wv * ga_g[i, sl]
                w_g[i, sl] = wv * rows_c[i, sl]
            return 0
        lax.fori_loop(0, CPS, _mq, 0)

        for i in range(CPS):
            pltpu.async_copy(w_g.at[i], p_sh.at[src_g.at[i]], sem_q,
                             add=True)
        _fire_g(rows_c, 2, sem_gc)

        def _body(t, _):
            ca = 3 * t
            _wait_g(rows_a, sem_ga)
            _scale_rows(rows_a, ca)
            dsa = _fire_s(rows_a, ca, sem_sa)

            _wait_g(rows_b, sem_gb)
            _scale_rows(rows_b, ca + 1)
            dsb = _fire_s(rows_b, ca + 1, sem_sb)

            dsa.wait()
            _fire_g(rows_a, ca + 3, sem_ga)

            _wait_g(rows_c, sem_gc)
            _scale_rows(rows_c, ca + 2)
            dsc = _fire_s(rows_c, ca + 2, sem_sc)

            dsb.wait()

            @pl.when(t < CPS // 3 - 1)
            def _():
                _fire_g(rows_b, ca + 4, sem_gb)

            dsc.wait()

            @pl.when(t < CPS // 3 - 1)
            def _():
                _fire_g(rows_c, ca + 5, sem_gc)
            return 0
        lax.fori_loop(0, CPS // 3, _body, 0)

        ct = CPS - 1
        _wait_g(rows_a, sem_ga)
        _scale_rows(rows_a, ct)
        _fire_s(rows_a, ct, sem_sa)
        _wait_s(rows_a, sem_sa)
        for i in range(CPS):
            pltpu.make_async_copy(
                w_g.at[0], p_sh.at[src_g.at[0]], sem_q).wait()
        return 0
    lax.fori_loop(0, NSLAB, _slab, 0)

    plsc.subcore_barrier()

    pltpu.sync_copy(b_sh.at[pl.ds(r0, NPT)], b_sl)

    def _brow_blk(rows_ref, base):
        def _brow(g, _):
            bv16 = b_sl[pl.ds(base + g * 16, 16)]
            for rr in range(16):
                r = g * 16 + rr
                bv = bv16[rr]
                for c in range(D // 16):
                    sl = pl.ds(c * 16, 16)
                    rows_ref[r, sl] = rows_ref[r, sl] * bv
            return 0
        lax.fori_loop(0, CK // 16, _brow, 0)

    def _zero_blk(rows_ref):
        def _z(j, _2):
            for c in range(D // 16):
                rows_ref[j, pl.ds(c * 16, 16)] = jnp.zeros((16,), jnp.float32)
            return 0
        lax.fori_loop(0, CK, _z, 0)

    def _fetch_blk(rows_ref, base, sem):
        @pl.when(r0 + base < N)
        def _():
            pltpu.async_copy(
                y_sh.at[pl.ds(r0 + base, CK), :], rows_ref, sem).wait()

        @pl.when(r0 + base >= N)
        def _():
            _zero_blk(rows_ref)

    def _epair(t, _):
        b0 = (2 * t) * CK
        b1 = (2 * t + 1) * CK

        @pl.when(t > 0)
        def _():
            pltpu.make_async_copy(
                rows_a, ys_hbm.at[cid, pl.ds(r0, CK), :], sem_ga).wait()
            pltpu.make_async_copy(
                rows_b, ys_hbm.at[cid, pl.ds(r0, CK), :], sem_gb).wait()
        _fetch_blk(rows_a, b0, sem_ga)
        _brow_blk(rows_a, b0)
        pltpu.async_copy(
            rows_a, ys_hbm.at[cid, pl.ds(r0 + b0, CK), :], sem_ga)
        _fetch_blk(rows_b, b1, sem_gb)
        _brow_blk(rows_b, b1)
        pltpu.async_copy(
            rows_b, ys_hbm.at[cid, pl.ds(r0 + b1, CK), :], sem_gb)
        return 0
    lax.fori_loop(0, RB // 2, _epair, 0)
    pltpu.make_async_copy(
        rows_a, ys_hbm.at[cid, pl.ds(r0, CK), :], sem_ga).wait()
    pltpu.make_async_copy(
        rows_b, ys_hbm.at[cid, pl.ds(r0, CK), :], sem_gb).wait()

    pltpu.sync_copy(p_sh.at[pl.ds(r0, NPT)], tmp_vm)

    pltpu.sync_copy(a_sh.at[pl.ds(r0, NPT)], b_sl)

    def _g(i, _):
        sl = pl.ds(i * 16, 16)
        tmp_vm[sl] = tmp_vm[sl] * b_sl[sl]
        return 0
    lax.fori_loop(0, NPT // 16, _g, 0)
    pltpu.sync_copy(tmp_vm, g_hbm.at[cid, pl.ds(r0, NPT)])


_sc_call = functools.partial(
    pl.kernel,
    out_type=(
        jax.ShapeDtypeStruct((NC, NP, D), jnp.float32),
        jax.ShapeDtypeStruct((NC, NP), jnp.float32),
    ),
    mesh=plsc.VectorSubcoreMesh(core_axis_name="c", subcore_axis_name="s"),
    scratch_types=[
        pltpu.VMEM((CPS, CK), jnp.int32),
        pltpu.VMEM((CPS, CK), jnp.int32),
        pltpu.VMEM((CPS, CK), jnp.float32),
        pltpu.VMEM((CPS, CK), jnp.float32),
        pltpu.VMEM((CK, D), jnp.float32),
        pltpu.VMEM((CK, D), jnp.float32),
        pltpu.VMEM((CK, D), jnp.float32),
        pltpu.VMEM((CK,), jnp.float32),
        pltpu.VMEM((NPT,), jnp.float32),
        pltpu.VMEM((NPT,), jnp.float32),
        pltpu.VMEM_SHARED((N, D), jnp.float32),
        pltpu.VMEM_SHARED((NP,), jnp.float32),
        pltpu.VMEM_SHARED((NP,), jnp.float32),
        pltpu.VMEM_SHARED((NP,), jnp.float32),
        pltpu.SemaphoreType.DMA,
        pltpu.SemaphoreType.DMA,
        pltpu.SemaphoreType.DMA,
        pltpu.SemaphoreType.DMA,
        pltpu.SemaphoreType.DMA,
        pltpu.SemaphoreType.DMA,
        pltpu.SemaphoreType.DMA,
    ],
)(_sc_body)


BLK = 1024
NBLK = NP // BLK


def _tc_body(ysf, ysl, gf, gl, w0f, w0l, b0f, b0l, w1f, w1l, b1f, b1l,
             out, acc):
    i = pl.program_id(0)

    @pl.when(i == 0)
    def _():
        acc[...] = jnp.zeros((2, D), jnp.float32)

    h = jnp.maximum(
        jnp.dot(ysf[...], w0f[...], preferred_element_type=jnp.float32)
        + jnp.dot(ysl[...], w0l[...], preferred_element_type=jnp.float32)
        + b0f[...] + b0l[...],
        0.0,
    )
    acc[0:1, :] += jnp.dot(gf[0], h, preferred_element_type=jnp.float32)
    acc[1:2, :] += jnp.dot(gl[0], h, preferred_element_type=jnp.float32)

    @pl.when(i == NBLK - 1)
    def _():
        out[...] = (
            jnp.dot(acc[0:1, :] * (1.0 / N), w1f[...],
                    preferred_element_type=jnp.float32)
            + jnp.dot(acc[1:2, :] * (1.0 / N), w1l[...],
                      preferred_element_type=jnp.float32)
            + b1f[...] + b1l[...]
        )


_tc_call = pl.pallas_call(
    _tc_body,
    grid=(NBLK,),
    in_specs=[
        pl.BlockSpec((BLK, D), lambda i: (i, 0)),
        pl.BlockSpec((BLK, D), lambda i: (i, 0)),
        pl.BlockSpec((1, 1, BLK), lambda i: (i, 0, 0)),
        pl.BlockSpec((1, 1, BLK), lambda i: (i, 0, 0)),
        pl.BlockSpec((D, D), lambda i: (0, 0)),
        pl.BlockSpec((D, D), lambda i: (0, 0)),
        pl.BlockSpec((1, D), lambda i: (0, 0)),
        pl.BlockSpec((1, D), lambda i: (0, 0)),
        pl.BlockSpec((D, D), lambda i: (0, 0)),
        pl.BlockSpec((D, D), lambda i: (0, 0)),
        pl.BlockSpec((1, D), lambda i: (0, 0)),
        pl.BlockSpec((1, D), lambda i: (0, 0)),
    ],
    out_specs=pl.BlockSpec((1, D), lambda i: (0, 0)),
    out_shape=jax.ShapeDtypeStruct((1, D), jnp.float32),
    scratch_shapes=[pltpu.VMEM((2, D), jnp.float32)],
)


def kernel(x, edge_index_follows, edge_weight_follows, edge_index_likes,
           edge_weight_likes, W0_follows, b0_follows, W0_likes, b0_likes,
           W1_follows, b1_follows, W1_likes, b1_likes):
    shp = (NS, NSLAB, CPS, CK)
    ys, g = _sc_call(
        x,
        edge_index_follows[0].reshape(shp), edge_index_follows[1].reshape(shp),
        edge_weight_follows.reshape(shp),
        edge_index_likes[0].reshape(shp), edge_index_likes[1].reshape(shp),
        edge_weight_likes.reshape(shp),
    )

    out = _tc_call(
        ys[0], ys[1],
        g[0].reshape(NBLK, 1, BLK), g[1].reshape(NBLK, 1, BLK),
        W0_follows, W0_likes,
        b0_follows.reshape(1, D), b0_likes.reshape(1, D),
        W1_follows, W1_likes,
        b1_follows.reshape(1, D), b1_likes.reshape(1, D),
    )
    return out
